# baseline (speedup 1.0000x reference)
import functools

import jax
import jax.numpy as jnp
from jax import lax
from jax.experimental import pallas as pl
from jax.experimental.pallas import tpu as pltpu
from jax.experimental.pallas import tpu_sc as plsc

_B = 4096
_QL = 20
_DL = 50
_WD = 100
_D = 128
_NWORD = 1000000
_VOCAB_C = 1000

_NC = 2
_NS = 16
_NW = _NC * _NS
_CHUNK = 64

_NQ = _B * _QL
_NDN = 2 * _B * _DL


def _proj_body(w_ref, wt_ref, b_ref, out_ref):
    w16 = w_ref[...].astype(jnp.bfloat16)
    wt16 = wt_ref[...].astype(jnp.bfloat16)
    out_ref[...] = jnp.dot(w16, wt16,
                           preferred_element_type=jnp.float32) + b_ref[...]


def _proj_call(rb):
    return pl.pallas_call(
        _proj_body,
        out_shape=jax.ShapeDtypeStruct((_NWORD, _D), jnp.float32),
        grid=(_NWORD // rb,),
        in_specs=[
            pl.BlockSpec((rb, _WD), lambda i: (i, 0)),
            pl.BlockSpec((_WD, _D), lambda i: (0, 0)),
            pl.BlockSpec((1, _D), lambda i: (0, 0)),
        ],
        out_specs=pl.BlockSpec((rb, _D), lambda i: (i, 0)),
    )


def _sc_body(widx_q, cidx_q, widx_dn, cidx_dn, table, cemb, out_q, out_dn,
             idx_v, cdx_v, w0, w1, w2, w3, c0, c1, c2, c3, spm,
             gw0, gw1, gw2, gw3, gc0, gc1, gc2, gc3, ss0, ss1, ss2, ss3):
    sid = lax.axis_index("s")
    cid = lax.axis_index("c")
    wid = sid * _NC + cid

    wbufs = (w0, w1, w2, w3)
    cbufs = (c0, c1, c2, c3)
    gwsems = (gw0, gw1, gw2, gw3)
    gcsems = (gc0, gc1, gc2, gc3)
    ssems = (ss0, ss1, ss2, ss3)

    @pl.when(sid == 0)
    def _():
        pltpu.sync_copy(cemb, spm)

    plsc.subcore_barrier()

    def add_chunk(p):
        def body(r, _):
            for g in range(_D // 16):
                sl = pl.ds(g * 16, 16)
                wbufs[p][r, sl] += cbufs[p][r, sl]
            return ()
        lax.fori_loop(0, _CHUNK, body, (), unroll=8)

    def seg(widx, cidx, out, n_per_w):
        base = wid * n_per_w
        nch = n_per_w // _CHUNK
        pltpu.sync_copy(widx.at[pl.ds(base, n_per_w)],
                        idx_v.at[pl.ds(0, n_per_w)])
        pltpu.sync_copy(cidx.at[pl.ds(base, n_per_w)],
                        cdx_v.at[pl.ds(0, n_per_w)])

        def issue_gathers(c, t):
            sl = pl.ds(c * _CHUNK, _CHUNK)
            pltpu.async_copy(table.at[idx_v.at[sl]], wbufs[t], gwsems[t])
            pltpu.async_copy(spm.at[cdx_v.at[sl]], cbufs[t], gcsems[t])

        def wait_gathers(c, t):
            sl = pl.ds(c * _CHUNK, _CHUNK)
            pltpu.make_async_copy(table.at[idx_v.at[sl]], wbufs[t],
                                  gwsems[t]).wait()
            pltpu.make_async_copy(spm.at[cdx_v.at[sl]], cbufs[t],
                                  gcsems[t]).wait()

        def issue_store(c, t):
            pltpu.async_copy(wbufs[t],
                             out.at[pl.ds(base + c * _CHUNK, _CHUNK)],
                             ssems[t])

        def wait_store(c, t):
            pltpu.make_async_copy(wbufs[t],
                                  out.at[pl.ds(base + c * _CHUNK, _CHUNK)],
                                  ssems[t]).wait()

        def process(c, t):
            wait_gathers(c, t)
            add_chunk(t)
            issue_store(c, t)

        def quad(j, _):
            for t in range(4):
                c = 4 * j + t

                @pl.when(j > 0)
                def _():
                    wait_store(c - 4, t)
                issue_gathers(c, t)
                if t >= 2:
                    process(c - 2, t - 2)
                else:
                    @pl.when(j > 0)
                    def _():
                        process(c - 2, (t - 2) % 4)
            return ()

        lax.fori_loop(0, nch // 4, quad, ())
        process(nch - 2, 2)
        process(nch - 1, 3)
        for t in range(4):
            wait_store(nch - 4 + t, t)

    seg(widx_q, cidx_q, out_q, _NQ // _NW)
    seg(widx_dn, cidx_dn, out_dn, _NDN // _NW)


@functools.cache
def _sc_gather_call():
    n_per_w = _NDN // _NW
    return pl.kernel(
        _sc_body,
        out_type=(
            jax.ShapeDtypeStruct((_NQ, _D), jnp.float32),
            jax.ShapeDtypeStruct((_NDN, _D), jnp.float32),
        ),
        mesh=plsc.VectorSubcoreMesh(core_axis_name="c", subcore_axis_name="s",
                                    num_cores=_NC),
        scratch_types=(
            [pltpu.VMEM((n_per_w,), jnp.int32)] * 2
            + [pltpu.VMEM((_CHUNK, _D), jnp.float32)] * 8
            + [pltpu.VMEM_SHARED((_VOCAB_C, _D), jnp.float32)]
            + [pltpu.SemaphoreType.DMA] * 12
        ),
    )


def _tower_body(rows_ref, mid_ref, attk_ref, atto_ref, out_ref, *, bb, seq):
    t = bb * seq
    x = rows_ref[...]
    th = jnp.tanh(lax.dot_general(x, attk_ref[...],
                                  (((1,), (1,)), ((), ())),
                                  preferred_element_type=jnp.float32))
    logits = jnp.sum(th * atto_ref[...], axis=1, keepdims=True)
    mask = (mid_ref[...] > 0).astype(jnp.float32)
    logits = logits - (1.0 - mask) * 1e12
    gmax = jnp.max(logits)
    ew = jnp.exp(logits - gmax)

    tdiv = lax.broadcasted_iota(jnp.int32, (bb, t), 1) // seq
    bidx = lax.broadcasted_iota(jnp.int32, (bb, t), 0)
    sel = (tdiv == bidx).astype(jnp.float32)
    xw = jnp.concatenate([x * ew, ew], axis=1)
    agg = jnp.dot(sel, xw, preferred_element_type=jnp.float32)
    den = jnp.maximum(agg[:, _D:_D + 1], 1e-30)
    pooled = agg[:, :_D] / den
    nrm = jnp.sqrt(jnp.sum(pooled * pooled, axis=1, keepdims=True))
    out_ref[...] = pooled / jnp.maximum(nrm, 1e-12)


def _tower_call(seq, bb, nb):
    t = bb * seq
    return pl.pallas_call(
        functools.partial(_tower_body, bb=bb, seq=seq),
        out_shape=jax.ShapeDtypeStruct((nb, _D), jnp.float32),
        grid=(nb // bb,),
        in_specs=[
            pl.BlockSpec((t, _D), lambda i: (i, 0)),
            pl.BlockSpec((t, 1), lambda i: (i, 0)),
            pl.BlockSpec((_D, _D), lambda i: (0, 0)),
            pl.BlockSpec((1, _D), lambda i: (0, 0)),
        ],
        out_specs=pl.BlockSpec((bb, _D), lambda i: (i, 0)),
    )


def _loss_body(q_ref, dn_ref, out_ref, *, rb):
    i = pl.program_id(0)
    qb = q_ref[...]
    s = 5.0 * lax.dot_general(qb, dn_ref[...], (((1,), (1,)), ((), ())),
                              preferred_element_type=jnp.float32)
    m = jnp.max(s, axis=1, keepdims=True)
    ssum = jnp.sum(jnp.exp(s - m), axis=1, keepdims=True)
    lse = m + jnp.log(ssum)
    dmatch = dn_ref[pl.ds(i * rb, rb), :]
    diag = 5.0 * jnp.sum(qb * dmatch, axis=1, keepdims=True)
    partial = jnp.sum(lse - diag) / _B

    @pl.when(i == 0)
    def _():
        out_ref[...] = jnp.zeros_like(out_ref)

    out_ref[...] += partial


def _loss_call(rb):
    return pl.pallas_call(
        functools.partial(_loss_body, rb=rb),
        out_shape=jax.ShapeDtypeStruct((1, 1), jnp.float32),
        grid=(_B // rb,),
        in_specs=[
            pl.BlockSpec((rb, _D), lambda i: (i, 0)),
            pl.BlockSpec((2 * _B, _D), lambda i: (0, 0)),
        ],
        out_specs=pl.BlockSpec((1, 1), lambda i: (0, 0)),
    )


def kernel(query, querychar, doc, docchar, neg, negchar, wordemb, charemb,
           w2c_w, w2c_b, qatt_k, qatt_o, datt_k, datt_o):
    iq = query.reshape(-1).astype(jnp.int32)
    idn = jnp.concatenate([doc.reshape(-1), neg.reshape(-1)]).astype(jnp.int32)
    cq = querychar.reshape(-1).astype(jnp.int32)
    cdn = jnp.concatenate([docchar.reshape(-1),
                           negchar.reshape(-1)]).astype(jnp.int32)

    ptable = _proj_call(20000)(wordemb, w2c_w.T, w2c_b.reshape(1, _D))
    qrows, dnrows = _sc_gather_call()(iq, cq, idn, cdn, ptable, charemb)

    qemb = _tower_call(_QL, 128, _B)(
        qrows, cq.reshape(-1, 1), qatt_k, qatt_o)
    dnemb = _tower_call(_DL, 128, 2 * _B)(
        dnrows, idn.reshape(-1, 1), datt_k, datt_o)

    loss = _loss_call(512)(qemb, dnemb)
    return loss.reshape(())


# speedup vs baseline: 1.0090x; 1.0090x over previous
import functools

import jax
import jax.numpy as jnp
from jax import lax
from jax.experimental import pallas as pl
from jax.experimental.pallas import tpu as pltpu
from jax.experimental.pallas import tpu_sc as plsc

_B = 4096
_QL = 20
_DL = 50
_WD = 100
_D = 128
_NWORD = 1000000
_VOCAB_C = 1000

_NC = 2
_NS = 16
_NW = _NC * _NS
_CHUNK = 64

_NQ = _B * _QL
_NDN = 2 * _B * _DL


def _proj_body(w_ref, wt_ref, b_ref, out_ref):
    w16 = w_ref[...].astype(jnp.bfloat16)
    wt16 = wt_ref[...].astype(jnp.bfloat16)
    out_ref[...] = jnp.dot(w16, wt16,
                           preferred_element_type=jnp.float32) + b_ref[...]


def _proj_call(rb):
    return pl.pallas_call(
        _proj_body,
        out_shape=jax.ShapeDtypeStruct((_NWORD, _D), jnp.float32),
        grid=(_NWORD // rb,),
        in_specs=[
            pl.BlockSpec((rb, _WD), lambda i: (i, 0)),
            pl.BlockSpec((_WD, _D), lambda i: (0, 0)),
            pl.BlockSpec((1, _D), lambda i: (0, 0)),
        ],
        out_specs=pl.BlockSpec((rb, _D), lambda i: (i, 0)),
    )


def _sc_body(widx_q, cidx_q, widx_dn, cidx_dn, table, cemb, out_q, out_dn,
             idx_v, cdx_v, w0, w1, w2, w3, c0, c1, c2, c3, spm,
             gw0, gw1, gw2, gw3, gc0, gc1, gc2, gc3, ss0, ss1, ss2, ss3):
    sid = lax.axis_index("s")
    cid = lax.axis_index("c")
    wid = sid * _NC + cid

    wbufs = (w0, w1, w2, w3)
    cbufs = (c0, c1, c2, c3)
    gwsems = (gw0, gw1, gw2, gw3)
    gcsems = (gc0, gc1, gc2, gc3)
    ssems = (ss0, ss1, ss2, ss3)

    @pl.when(sid == 0)
    def _():
        pltpu.sync_copy(cemb, spm)

    plsc.subcore_barrier()

    def add_chunk(p):
        def body(r, _):
            for g in range(_D // 16):
                sl = pl.ds(g * 16, 16)
                wbufs[p][r, sl] += cbufs[p][r, sl]
            return ()
        lax.fori_loop(0, _CHUNK, body, (), unroll=4)

    def seg(widx, cidx, out, n_per_w):
        base = wid * n_per_w
        nch = n_per_w // _CHUNK
        pltpu.sync_copy(widx.at[pl.ds(base, n_per_w)],
                        idx_v.at[pl.ds(0, n_per_w)])
        pltpu.sync_copy(cidx.at[pl.ds(base, n_per_w)],
                        cdx_v.at[pl.ds(0, n_per_w)])

        def issue_gathers(c, t):
            sl = pl.ds(c * _CHUNK, _CHUNK)
            pltpu.async_copy(table.at[idx_v.at[sl]], wbufs[t], gwsems[t])
            pltpu.async_copy(spm.at[cdx_v.at[sl]], cbufs[t], gcsems[t])

        def wait_gathers(c, t):
            sl = pl.ds(c * _CHUNK, _CHUNK)
            pltpu.make_async_copy(table.at[idx_v.at[sl]], wbufs[t],
                                  gwsems[t]).wait()
            pltpu.make_async_copy(spm.at[cdx_v.at[sl]], cbufs[t],
                                  gcsems[t]).wait()

        def issue_store(c, t):
            pltpu.async_copy(wbufs[t],
                             out.at[pl.ds(base + c * _CHUNK, _CHUNK)],
                             ssems[t])

        def wait_store(c, t):
            pltpu.make_async_copy(wbufs[t],
                                  out.at[pl.ds(base + c * _CHUNK, _CHUNK)],
                                  ssems[t]).wait()

        def process(c, t):
            wait_gathers(c, t)
            add_chunk(t)
            issue_store(c, t)

        def quad(j, _):
            for t in range(4):
                c = 4 * j + t

                @pl.when(j > 0)
                def _():
                    wait_store(c - 4, t)
                issue_gathers(c, t)
                if t >= 2:
                    process(c - 2, t - 2)
                else:
                    @pl.when(j > 0)
                    def _():
                        process(c - 2, (t - 2) % 4)
            return ()

        lax.fori_loop(0, nch // 4, quad, ())
        process(nch - 2, 2)
        process(nch - 1, 3)
        for t in range(4):
            wait_store(nch - 4 + t, t)

    seg(widx_q, cidx_q, out_q, _NQ // _NW)
    seg(widx_dn, cidx_dn, out_dn, _NDN // _NW)


@functools.cache
def _sc_gather_call():
    n_per_w = _NDN // _NW
    return pl.kernel(
        _sc_body,
        out_type=(
            jax.ShapeDtypeStruct((_NQ, _D), jnp.float32),
            jax.ShapeDtypeStruct((_NDN, _D), jnp.float32),
        ),
        mesh=plsc.VectorSubcoreMesh(core_axis_name="c", subcore_axis_name="s",
                                    num_cores=_NC),
        scratch_types=(
            [pltpu.VMEM((n_per_w,), jnp.int32)] * 2
            + [pltpu.VMEM((_CHUNK, _D), jnp.float32)] * 8
            + [pltpu.VMEM_SHARED((_VOCAB_C, _D), jnp.float32)]
            + [pltpu.SemaphoreType.DMA] * 12
        ),
    )


def _tower_body(rows_ref, mid_ref, attk_ref, atto_ref, out_ref, *, bb, seq):
    t = bb * seq
    x = rows_ref[...]
    th = jnp.tanh(lax.dot_general(x, attk_ref[...],
                                  (((1,), (1,)), ((), ())),
                                  preferred_element_type=jnp.float32))
    logits = jnp.sum(th * atto_ref[...], axis=1, keepdims=True)
    mask = (mid_ref[...] > 0).astype(jnp.float32)
    logits = logits - (1.0 - mask) * 1e12
    gmax = jnp.max(logits)
    ew = jnp.exp(logits - gmax)

    tdiv = lax.broadcasted_iota(jnp.int32, (bb, t), 1) // seq
    bidx = lax.broadcasted_iota(jnp.int32, (bb, t), 0)
    sel = (tdiv == bidx).astype(jnp.float32)
    xw = jnp.concatenate([x * ew, ew], axis=1)
    agg = jnp.dot(sel, xw, preferred_element_type=jnp.float32)
    den = jnp.maximum(agg[:, _D:_D + 1], 1e-30)
    pooled = agg[:, :_D] / den
    nrm = jnp.sqrt(jnp.sum(pooled * pooled, axis=1, keepdims=True))
    out_ref[...] = pooled / jnp.maximum(nrm, 1e-12)


def _tower_call(seq, bb, nb):
    t = bb * seq
    return pl.pallas_call(
        functools.partial(_tower_body, bb=bb, seq=seq),
        out_shape=jax.ShapeDtypeStruct((nb, _D), jnp.float32),
        grid=(nb // bb,),
        in_specs=[
            pl.BlockSpec((t, _D), lambda i: (i, 0)),
            pl.BlockSpec((t, 1), lambda i: (i, 0)),
            pl.BlockSpec((_D, _D), lambda i: (0, 0)),
            pl.BlockSpec((1, _D), lambda i: (0, 0)),
        ],
        out_specs=pl.BlockSpec((bb, _D), lambda i: (i, 0)),
    )


def _loss_body(q_ref, dn_ref, out_ref, *, rb):
    i = pl.program_id(0)
    qb = q_ref[...]
    s = 5.0 * lax.dot_general(qb, dn_ref[...], (((1,), (1,)), ((), ())),
                              preferred_element_type=jnp.float32)
    m = jnp.max(s, axis=1, keepdims=True)
    ssum = jnp.sum(jnp.exp(s - m), axis=1, keepdims=True)
    lse = m + jnp.log(ssum)
    dmatch = dn_ref[pl.ds(i * rb, rb), :]
    diag = 5.0 * jnp.sum(qb * dmatch, axis=1, keepdims=True)
    partial = jnp.sum(lse - diag) / _B

    @pl.when(i == 0)
    def _():
        out_ref[...] = jnp.zeros_like(out_ref)

    out_ref[...] += partial


def _loss_call(rb):
    return pl.pallas_call(
        functools.partial(_loss_body, rb=rb),
        out_shape=jax.ShapeDtypeStruct((1, 1), jnp.float32),
        grid=(_B // rb,),
        in_specs=[
            pl.BlockSpec((rb, _D), lambda i: (i, 0)),
            pl.BlockSpec((2 * _B, _D), lambda i: (0, 0)),
        ],
        out_specs=pl.BlockSpec((1, 1), lambda i: (0, 0)),
    )


def kernel(query, querychar, doc, docchar, neg, negchar, wordemb, charemb,
           w2c_w, w2c_b, qatt_k, qatt_o, datt_k, datt_o):
    iq = query.reshape(-1).astype(jnp.int32)
    idn = jnp.concatenate([doc.reshape(-1), neg.reshape(-1)]).astype(jnp.int32)
    cq = querychar.reshape(-1).astype(jnp.int32)
    cdn = jnp.concatenate([docchar.reshape(-1),
                           negchar.reshape(-1)]).astype(jnp.int32)

    ptable = _proj_call(20000)(wordemb, w2c_w.T, w2c_b.reshape(1, _D))
    qrows, dnrows = _sc_gather_call()(iq, cq, idn, cdn, ptable, charemb)

    qemb = _tower_call(_QL, 128, _B)(
        qrows, cq.reshape(-1, 1), qatt_k, qatt_o)
    dnemb = _tower_call(_DL, 128, 2 * _B)(
        dnrows, idn.reshape(-1, 1), datt_k, datt_o)

    loss = _loss_call(512)(qemb, dnemb)
    return loss.reshape(())
